# baseline (speedup 1.0000x reference)
import functools

import jax
import jax.numpy as jnp
from jax import lax
from jax.experimental import pallas as pl
from jax.experimental.pallas import tpu as pltpu
from jax.experimental.pallas import tpu_sc as plsc

NBASIS = 1000000
NUNIQ = 65536

_NC = 2
_NS = 16
_NW = _NC * _NS
_LANES = 16

_FULL = 31264
_TAIL = NBASIS - (_NW - 1) * _FULL
_LO = 40960

_mesh = plsc.VectorSubcoreMesh(core_axis_name="c", subcore_axis_name="s")


@functools.partial(
    pl.kernel,
    mesh=_mesh,
    out_type=jax.ShapeDtypeStruct((NBASIS,), jnp.float32),
    scratch_types=[
        pltpu.VMEM((NUNIQ,), jnp.float32),
        pltpu.VMEM((_FULL,), jnp.int32),
        pltpu.VMEM((_FULL,), jnp.float32),
        pltpu.VMEM((_LANES,), jnp.float32),
        pltpu.VMEM_SHARED((_LO,), jnp.float32),
        pltpu.SemaphoreType.DMA,
        pltpu.SemaphoreType.DMA,
        pltpu.SemaphoreType.DMA,
        pltpu.SemaphoreType.DMA,
    ],
    compiler_params=pltpu.CompilerParams(
        needs_layout_passes=False,
        disable_bounds_checks=True,
        disable_semaphore_checks=True,
        skip_device_barrier=True,
    ),
)
def _sc_gather(ke_hbm, idx_hbm, step_hbm, out_hbm,
               tab_v, idx_v, out_v, step_v, tab_lo_sh,
               sem_tab, sem_in, sem_out, sem_stage):
    c = lax.axis_index("c")
    s = lax.axis_index("s")
    wid = s * _NC + c
    base = wid * _FULL

    @pl.when(s == 0)
    def _():
        pltpu.async_copy(ke_hbm.at[pl.ds(0, _LO)], tab_lo_sh, sem_stage).wait()

    tab_cp = pltpu.async_copy(
        ke_hbm.at[pl.ds(_LO, NUNIQ - _LO)],
        tab_v.at[pl.ds(_LO, NUNIQ - _LO)],
        sem_tab,
    )
    pltpu.sync_copy(step_hbm, step_v)
    plsc.subcore_barrier()
    pltpu.sync_copy(tab_lo_sh, tab_v.at[pl.ds(0, _LO)])

    def work(n):
        half = (n // (2 * _LANES)) * _LANES
        sizes = (half, n - half)
        offs = (0, half)
        in_cps = [
            pltpu.async_copy(
                idx_hbm.at[pl.ds(base + o, sz)], idx_v.at[pl.ds(o, sz)], sem_in
            )
            for o, sz in zip(offs, sizes)
        ]
        tab_cp.wait()
        sv = step_v[...]
        out_cps = []
        for k in range(2):
            in_cps[k].wait()
            o = offs[k]

            @plsc.parallel_loop(0, sizes[k] // _LANES, unroll=8)
            def body(i, o=o):
                off = pl.multiple_of(o + i * _LANES, _LANES)
                iv = idx_v[pl.ds(off, _LANES)]
                out_v[pl.ds(off, _LANES)] = plsc.load_gather(tab_v, [iv]) * sv

            out_cps.append(
                pltpu.async_copy(
                    out_v.at[pl.ds(o, sizes[k])],
                    out_hbm.at[pl.ds(base + o, sizes[k])],
                    sem_out,
                )
            )
        for cp in out_cps:
            cp.wait()

    @pl.when(wid != _NW - 1)
    def _():
        work(_FULL)

    @pl.when(wid == _NW - 1)
    def _():
        work(_TAIL)


def kernel(ke, ke_invidx, step):
    idx = ke_invidx.astype(jnp.int32)
    step_vec = jnp.full((_LANES,), step, dtype=jnp.float32)
    return _sc_gather(ke, idx, step_vec)


# speedup vs baseline: 1.0300x; 1.0300x over previous
import functools

import jax
import jax.numpy as jnp
from jax import lax
from jax.experimental import pallas as pl
from jax.experimental.pallas import tpu as pltpu
from jax.experimental.pallas import tpu_sc as plsc

NBASIS = 1000000
NUNIQ = 65536

_NC = 2
_NS = 16
_NW = _NC * _NS
_LANES = 16

_FULL = 31264
_TAIL = NBASIS - (_NW - 1) * _FULL
_LO = 40960
_C0 = 15408

_mesh = plsc.VectorSubcoreMesh(core_axis_name="c", subcore_axis_name="s")


@functools.partial(
    pl.kernel,
    mesh=_mesh,
    out_type=jax.ShapeDtypeStruct((NBASIS,), jnp.float32),
    scratch_types=[
        pltpu.VMEM((NUNIQ,), jnp.float32),
        pltpu.VMEM((_FULL,), jnp.int32),
        pltpu.VMEM((_FULL,), jnp.float32),
        pltpu.VMEM((_LANES,), jnp.float32),
        pltpu.VMEM_SHARED((_LO,), jnp.float32),
        pltpu.SemaphoreType.DMA,
        pltpu.SemaphoreType.DMA,
        pltpu.SemaphoreType.DMA,
        pltpu.SemaphoreType.DMA,
    ],
    compiler_params=pltpu.CompilerParams(needs_layout_passes=False),
)
def _sc_gather(ke_hbm, idx_hbm, step_hbm, out_hbm,
               tab_v, idx_v, out_v, step_v, tab_lo_sh,
               sem_tab, sem_in, sem_out, sem_stage):
    c = lax.axis_index("c")
    s = lax.axis_index("s")
    wid = s * _NC + c
    base = wid * _FULL

    in0_cp = pltpu.async_copy(
        idx_hbm.at[pl.ds(base, _C0)], idx_v.at[pl.ds(0, _C0)], sem_in
    )

    @pl.when(s == 0)
    def _():
        pltpu.async_copy(ke_hbm.at[pl.ds(0, _LO)], tab_lo_sh, sem_stage).wait()

    tab_cp = pltpu.async_copy(
        ke_hbm.at[pl.ds(_LO, NUNIQ - _LO)],
        tab_v.at[pl.ds(_LO, NUNIQ - _LO)],
        sem_tab,
    )
    pltpu.sync_copy(step_hbm, step_v)
    plsc.subcore_barrier()
    pltpu.sync_copy(tab_lo_sh, tab_v.at[pl.ds(0, _LO)])

    def work(n):
        sizes = (_C0, n - _C0)
        offs = (0, _C0)
        in_cps = [
            in0_cp,
            pltpu.async_copy(
                idx_hbm.at[pl.ds(base + _C0, n - _C0)],
                idx_v.at[pl.ds(_C0, n - _C0)],
                sem_in,
            ),
        ]
        tab_cp.wait()
        sv = step_v[...]
        out_cps = []
        for k in range(2):
            in_cps[k].wait()
            o = offs[k]

            @plsc.parallel_loop(0, sizes[k] // _LANES, unroll=8)
            def body(i, o=o):
                off = pl.multiple_of(o + i * _LANES, _LANES)
                iv = idx_v[pl.ds(off, _LANES)]
                out_v[pl.ds(off, _LANES)] = plsc.load_gather(tab_v, [iv]) * sv

            out_cps.append(
                pltpu.async_copy(
                    out_v.at[pl.ds(o, sizes[k])],
                    out_hbm.at[pl.ds(base + o, sizes[k])],
                    sem_out,
                )
            )
        for cp in out_cps:
            cp.wait()

    @pl.when(wid != _NW - 1)
    def _():
        work(_FULL)

    @pl.when(wid == _NW - 1)
    def _():
        work(_TAIL)


def kernel(ke, ke_invidx, step):
    idx = ke_invidx.astype(jnp.int32)
    step_vec = jnp.full((_LANES,), step, dtype=jnp.float32)
    return _sc_gather(ke, idx, step_vec)
